# baseline (speedup 1.0000x reference)
import functools

import jax
import jax.numpy as jnp
from jax import lax
from jax.experimental import pallas as pl
from jax.experimental.pallas import tpu as pltpu
from jax.experimental.pallas import tpu_sc as plsc

N_NODES = 10000
N_EDGES = 320000
D_IN = 128
D_HID = 128
D_OUT = 64

NP = 10240
NC = 2
NS = 16
NW = NC * NS
K = 128
CH = 80
CHP = CH + 2
EP = NW * CHP * K
ROWS_PER_TILE = NP // NS

_mesh = functools.partial(
    plsc.VectorSubcoreMesh,
    core_axis_name="c", subcore_axis_name="s",
    num_cores=NC, num_subcores=NS)


def _make_deg_kernel():
    ZR = 16

    @functools.partial(
        pl.kernel,
        out_type=jax.ShapeDtypeStruct((NC, NP, 128), jnp.float32),
        mesh=_mesh(),
        scratch_types=[
            pltpu.VMEM((CHP, K), jnp.int32),
            pltpu.VMEM((K, 128), jnp.float32),
            pltpu.VMEM((ZR, 128), jnp.float32),
            pltpu.VMEM_SHARED((NP, 128), jnp.float32),
            pltpu.SemaphoreType.DMA,
        ],
    )
    def deg_kernel(dst_hbm, out_hbm, dst_v, ones_v, zb_v, acc_s, ssem):
        cid = lax.axis_index("c")
        sid = lax.axis_index("s")
        wid = sid * NC + cid
        pltpu.sync_copy(dst_hbm.at[wid], dst_v)
        one = jnp.ones((16,), jnp.float32)
        zero = jnp.zeros((16,), jnp.float32)
        for r in range(K):
            for c in range(128 // 16):
                ones_v[r, pl.ds(c * 16, 16)] = one
        for r in range(ZR):
            for c in range(128 // 16):
                zb_v[r, pl.ds(c * 16, 16)] = zero
        row0 = sid * ROWS_PER_TILE

        def zloop(t, carry):
            pltpu.sync_copy(zb_v, acc_s.at[pl.ds(row0 + t * ZR, ZR)])
            return carry

        lax.fori_loop(jnp.int32(0), jnp.int32(ROWS_PER_TILE // ZR), zloop,
                      jnp.int32(0))
        plsc.subcore_barrier()

        def chunk(j, carry):
            pltpu.sync_copy(ones_v, acc_s.at[dst_v.at[j]], add=True)
            return carry

        lax.fori_loop(jnp.int32(0), jnp.int32(CH), chunk, jnp.int32(0))
        plsc.subcore_barrier()
        pltpu.sync_copy(acc_s.at[pl.ds(row0, ROWS_PER_TILE)],
                        out_hbm.at[cid, pl.ds(row0, ROWS_PER_TILE)])

    return deg_kernel


def _make_agg_kernel(D):
    ZR = 8

    @functools.partial(
        pl.kernel,
        out_type=jax.ShapeDtypeStruct((NC, NP, D), jnp.float32),
        mesh=_mesh(),
        scratch_types=[
            pltpu.VMEM((K,), jnp.int32),
            pltpu.VMEM((K,), jnp.int32),
            pltpu.VMEM((K,), jnp.int32),
            pltpu.VMEM((K,), jnp.int32),
            pltpu.VMEM((K, D), jnp.float32),
            pltpu.VMEM((K, D), jnp.float32),
            pltpu.VMEM((ZR, D), jnp.float32),
            pltpu.VMEM_SHARED((NP, D), jnp.float32),
            pltpu.SemaphoreType.DMA,
            pltpu.SemaphoreType.DMA,
            pltpu.SemaphoreType.DMA,
            pltpu.SemaphoreType.DMA,
            pltpu.SemaphoreType.DMA,
        ],
    )
    def agg_kernel(g_hbm, src_hbm, dst_hbm, out_hbm,
                   sidx0, sidx1, didx0, didx1, rows0, rows1, zb_v, acc_s,
                   isem0, isem1, gsem, ssem0, ssem1):
        cid = lax.axis_index("c")
        sid = lax.axis_index("s")
        wid = sid * NC + cid
        sidx = (sidx0, sidx1)
        didx = (didx0, didx1)
        rows = (rows0, rows1)
        isem = (isem0, isem1)
        ssem = (ssem0, ssem1)
        zero = jnp.zeros((16,), jnp.float32)
        for r in range(ZR):
            for c in range(D // 16):
                zb_v[r, pl.ds(c * 16, 16)] = zero
        row0 = sid * ROWS_PER_TILE

        def zloop(t, carry):
            pltpu.sync_copy(zb_v, acc_s.at[pl.ds(row0 + t * ZR, ZR)])
            return carry

        lax.fori_loop(jnp.int32(0), jnp.int32(ROWS_PER_TILE // ZR), zloop,
                      jnp.int32(0))
        plsc.subcore_barrier()

        def fetch_idx(t, e):
            pltpu.async_copy(src_hbm.at[wid, t], sidx[e], isem[e])
            pltpu.async_copy(dst_hbm.at[wid, t], didx[e], isem[e])

        def wait_idx(t, e):
            pltpu.make_async_copy(src_hbm.at[wid, t], sidx[e],
                                  isem[e]).wait()
            pltpu.make_async_copy(dst_hbm.at[wid, t], didx[e],
                                  isem[e]).wait()

        fetch_idx(jnp.int32(0), 0)
        wait_idx(jnp.int32(0), 0)
        pltpu.async_copy(g_hbm.at[sidx[0]], rows[0], gsem)
        fetch_idx(jnp.int32(1), 1)
        pltpu.make_async_copy(g_hbm.at[sidx[0]], rows[0], gsem).wait()

        def pipestep(u, carry):
            for e in range(2):
                t = u * 2 + e
                o = 1 - e
                pltpu.async_copy(rows[e], acc_s.at[didx[e]], ssem[e],
                                 add=True)
                wait_idx(t + 1, o)
                pltpu.async_copy(g_hbm.at[sidx[o]], rows[o], gsem)
                pltpu.make_async_copy(rows[e], acc_s.at[didx[e]],
                                      ssem[e]).wait()
                fetch_idx(t + 2, e)
                pltpu.make_async_copy(g_hbm.at[sidx[o]], rows[o],
                                      gsem).wait()
            return carry

        lax.fori_loop(jnp.int32(0), jnp.int32(CH // 2), pipestep,
                      jnp.int32(0))
        wait_idx(jnp.int32(CH + 1), 1)
        plsc.subcore_barrier()
        pltpu.sync_copy(acc_s.at[pl.ds(row0, ROWS_PER_TILE)],
                        out_hbm.at[cid, pl.ds(row0, ROWS_PER_TILE)])

    return agg_kernel


_deg_kernel = _make_deg_kernel()
_agg128 = _make_agg_kernel(D_HID)


def _tc_scale_in(x_pad, W1, degp):
    def body(x_ref, w_ref, degp_ref, g_ref, dis_ref):
        deg = degp_ref[0, :, 0:1] + degp_ref[1, :, 0:1] + 1.0
        dis = lax.rsqrt(deg)
        h = jnp.dot(x_ref[...], w_ref[...],
                    preferred_element_type=jnp.float32)
        g_ref[...] = h * dis
        dis_ref[...] = dis

    return pl.pallas_call(
        body,
        out_shape=(jax.ShapeDtypeStruct((NP, D_HID), jnp.float32),
                   jax.ShapeDtypeStruct((NP, 1), jnp.float32)),
    )(x_pad, W1, degp)


def _tc_mid(p1, g1, dis, b1, W2):
    def body(p_ref, g1_ref, dis_ref, b1_ref, w2_ref, g2_ref):
        dis = dis_ref[...]
        s = dis * (p_ref[0] + p_ref[1] + g1_ref[...]) + b1_ref[...]
        h = jnp.maximum(s, 0.0)
        g2_ref[...] = dis * jnp.dot(h, w2_ref[...],
                                    preferred_element_type=jnp.float32)

    return pl.pallas_call(
        body,
        out_shape=jax.ShapeDtypeStruct((NP, D_HID), jnp.float32),
    )(p1, g1, dis, b1, W2)


def _tc_out(p2, g2, dis, b2):
    def body(p_ref, g2_ref, dis_ref, b2_ref, z_ref):
        z_ref[...] = dis_ref[...] * (
            p_ref[0, :, :D_OUT] + p_ref[1, :, :D_OUT] + g2_ref[:, :D_OUT]
        ) + b2_ref[...]

    return pl.pallas_call(
        body,
        out_shape=jax.ShapeDtypeStruct((NP, D_OUT), jnp.float32),
    )(p2, g2, dis, b2)


def kernel(x, edge_index, W1, b1, W2, b2):
    ei = edge_index.astype(jnp.int32)
    pad = jnp.full((EP - N_EDGES,), NP - 1, jnp.int32)
    src3 = jnp.concatenate([ei[0], pad]).reshape(CHP, NW, K).transpose(1, 0, 2)
    dst3 = jnp.concatenate([ei[1], pad]).reshape(CHP, NW, K).transpose(1, 0, 2)
    x_pad = jnp.zeros((NP, D_IN), jnp.float32).at[:N_NODES].set(
        x.astype(jnp.float32))
    b1r = b1.astype(jnp.float32).reshape(1, D_HID)
    b2r = b2.astype(jnp.float32).reshape(1, D_OUT)

    W2p = jnp.zeros((D_HID, D_HID), jnp.float32).at[:, :D_OUT].set(
        W2.astype(jnp.float32))

    degp = _deg_kernel(dst3)
    g1, dis = _tc_scale_in(x_pad, W1.astype(jnp.float32), degp)
    p1 = _agg128(g1, src3, dst3)
    g2 = _tc_mid(p1, g1, dis, b1r, W2p)
    p2 = _agg128(g2, src3, dst3)
    z = _tc_out(p2, g2, dis, b2r)
    return z[:N_NODES].astype(jnp.float64)


# speedup vs baseline: 1.1969x; 1.1969x over previous
import functools

import jax
import jax.numpy as jnp
from jax import lax
from jax.experimental import pallas as pl
from jax.experimental.pallas import tpu as pltpu
from jax.experimental.pallas import tpu_sc as plsc

N_NODES = 10000
N_EDGES = 320000
D_IN = 128
D_HID = 128
D_OUT = 64

NP = 10240
NC = 2
NS = 16
NW = NC * NS
K = 128
CH = 80
CHP = CH + 2
EP = NW * CHP * K
ROWS_PER_TILE = NP // NS

_mesh = functools.partial(
    plsc.VectorSubcoreMesh,
    core_axis_name="c", subcore_axis_name="s",
    num_cores=NC, num_subcores=NS)


def _make_deg_kernel():
    ZR = 16

    @functools.partial(
        pl.kernel,
        out_type=jax.ShapeDtypeStruct((NC, NP, 128), jnp.float32),
        mesh=_mesh(),
        scratch_types=[
            pltpu.VMEM((CHP, K), jnp.int32),
            pltpu.VMEM((K, 128), jnp.float32),
            pltpu.VMEM((ZR, 128), jnp.float32),
            pltpu.VMEM_SHARED((NP, 128), jnp.float32),
            pltpu.SemaphoreType.DMA,
        ],
    )
    def deg_kernel(dst_hbm, out_hbm, dst_v, ones_v, zb_v, acc_s, ssem):
        cid = lax.axis_index("c")
        sid = lax.axis_index("s")
        wid = sid * NC + cid
        pltpu.sync_copy(dst_hbm.at[wid], dst_v)
        one = jnp.ones((16,), jnp.float32)
        zero = jnp.zeros((16,), jnp.float32)
        for r in range(K):
            for c in range(128 // 16):
                ones_v[r, pl.ds(c * 16, 16)] = one
        for r in range(ZR):
            for c in range(128 // 16):
                zb_v[r, pl.ds(c * 16, 16)] = zero
        row0 = sid * ROWS_PER_TILE

        def zloop(t, carry):
            pltpu.sync_copy(zb_v, acc_s.at[pl.ds(row0 + t * ZR, ZR)])
            return carry

        lax.fori_loop(jnp.int32(0), jnp.int32(ROWS_PER_TILE // ZR), zloop,
                      jnp.int32(0))
        plsc.subcore_barrier()

        def chunk(j, carry):
            pltpu.sync_copy(ones_v, acc_s.at[dst_v.at[j]], add=True)
            return carry

        lax.fori_loop(jnp.int32(0), jnp.int32(CH), chunk, jnp.int32(0))
        plsc.subcore_barrier()
        pltpu.sync_copy(acc_s.at[pl.ds(row0, ROWS_PER_TILE)],
                        out_hbm.at[cid, pl.ds(row0, ROWS_PER_TILE)])

    return deg_kernel


def _make_agg_kernel(D):
    ZR = 8
    NG = 4
    KH = K // NG

    @functools.partial(
        pl.kernel,
        out_type=jax.ShapeDtypeStruct((NC, NP, D), jnp.float32),
        mesh=_mesh(),
        scratch_types=[
            pltpu.VMEM((CHP, K), jnp.int32),
            pltpu.VMEM((CHP, K), jnp.int32),
            pltpu.VMEM((K, D), jnp.float32),
            pltpu.VMEM((ZR, D), jnp.float32),
            pltpu.VMEM_SHARED((NP, D), jnp.float32),
            pltpu.SemaphoreType.DMA,
        ],
    )
    def agg_kernel(g_hbm, src_hbm, dst_hbm, out_hbm,
                   src_v, dst_v, rows_v, zb_v, acc_s, gsem):
        cid = lax.axis_index("c")
        sid = lax.axis_index("s")
        wid = sid * NC + cid
        pltpu.sync_copy(src_hbm.at[wid], src_v)
        pltpu.sync_copy(dst_hbm.at[wid], dst_v)
        zero = jnp.zeros((16,), jnp.float32)
        for r in range(ZR):
            for c in range(D // 16):
                zb_v[r, pl.ds(c * 16, 16)] = zero
        row0 = sid * ROWS_PER_TILE

        def zloop(t, carry):
            pltpu.sync_copy(zb_v, acc_s.at[pl.ds(row0 + t * ZR, ZR)])
            return carry

        lax.fori_loop(jnp.int32(0), jnp.int32(ROWS_PER_TILE // ZR),
                      zloop, jnp.int32(0))
        plsc.subcore_barrier()

        def chunk(j, carry):
            for h in range(NG):
                pltpu.async_copy(
                    g_hbm.at[src_v.at[j, pl.ds(h * KH, KH)]],
                    rows_v.at[pl.ds(h * KH, KH)], gsem)
            for h in range(NG):
                pltpu.make_async_copy(
                    g_hbm.at[src_v.at[j, pl.ds(h * KH, KH)]],
                    rows_v.at[pl.ds(h * KH, KH)], gsem).wait()
            pltpu.sync_copy(rows_v, acc_s.at[dst_v.at[j]], add=True)
            return carry

        lax.fori_loop(jnp.int32(0), jnp.int32(CH), chunk, jnp.int32(0))
        plsc.subcore_barrier()
        pltpu.sync_copy(acc_s.at[pl.ds(row0, ROWS_PER_TILE)],
                        out_hbm.at[cid, pl.ds(row0, ROWS_PER_TILE)])

    return agg_kernel


_deg_kernel = _make_deg_kernel()
_agg128 = _make_agg_kernel(D_HID)


def _tc_scale_in(x_pad, W1, degp):
    def body(x_ref, w_ref, degp_ref, g_ref, dis_ref):
        deg = degp_ref[0, :, 0:1] + degp_ref[1, :, 0:1] + 1.0
        dis = lax.rsqrt(deg)
        h = jnp.dot(x_ref[...], w_ref[...],
                    preferred_element_type=jnp.float32)
        g_ref[...] = h * dis
        dis_ref[...] = dis

    return pl.pallas_call(
        body,
        out_shape=(jax.ShapeDtypeStruct((NP, D_HID), jnp.float32),
                   jax.ShapeDtypeStruct((NP, 1), jnp.float32)),
    )(x_pad, W1, degp)


def _tc_mid(p1, g1, dis, b1, W2):
    def body(p_ref, g1_ref, dis_ref, b1_ref, w2_ref, g2_ref):
        dis = dis_ref[...]
        s = dis * (p_ref[0] + p_ref[1] + g1_ref[...]) + b1_ref[...]
        h = jnp.maximum(s, 0.0)
        g2_ref[...] = dis * jnp.dot(h, w2_ref[...],
                                    preferred_element_type=jnp.float32)

    return pl.pallas_call(
        body,
        out_shape=jax.ShapeDtypeStruct((NP, D_HID), jnp.float32),
    )(p1, g1, dis, b1, W2)


def _tc_out(p2, g2, dis, b2):
    def body(p_ref, g2_ref, dis_ref, b2_ref, z_ref):
        z_ref[...] = dis_ref[...] * (
            p_ref[0, :, :D_OUT] + p_ref[1, :, :D_OUT] + g2_ref[:, :D_OUT]
        ) + b2_ref[...]

    return pl.pallas_call(
        body,
        out_shape=jax.ShapeDtypeStruct((NP, D_OUT), jnp.float32),
    )(p2, g2, dis, b2)


def kernel(x, edge_index, W1, b1, W2, b2):
    ei = edge_index.astype(jnp.int32)
    pad = jnp.full((EP - N_EDGES,), NP - 1, jnp.int32)
    src3 = jnp.concatenate([ei[0], pad]).reshape(CHP, NW, K).transpose(1, 0, 2)
    dst3 = jnp.concatenate([ei[1], pad]).reshape(CHP, NW, K).transpose(1, 0, 2)
    x_pad = jnp.zeros((NP, D_IN), jnp.float32).at[:N_NODES].set(
        x.astype(jnp.float32))
    b1r = b1.astype(jnp.float32).reshape(1, D_HID)
    b2r = b2.astype(jnp.float32).reshape(1, D_OUT)

    W2p = jnp.zeros((D_HID, D_HID), jnp.float32).at[:, :D_OUT].set(
        W2.astype(jnp.float32))

    degp = _deg_kernel(dst3)
    g1, dis = _tc_scale_in(x_pad, W1.astype(jnp.float32), degp)
    p1 = _agg128(g1, src3, dst3)
    g2 = _tc_mid(p1, g1, dis, b1r, W2p)
    p2 = _agg128(g2, src3, dst3)
    z = _tc_out(p2, g2, dis, b2r)
    return z[:N_NODES].astype(jnp.float64)


# speedup vs baseline: 1.2037x; 1.0057x over previous
import functools

import jax
import jax.numpy as jnp
from jax import lax
from jax.experimental import pallas as pl
from jax.experimental.pallas import tpu as pltpu
from jax.experimental.pallas import tpu_sc as plsc

N_NODES = 10000
N_EDGES = 320000
D_IN = 128
D_HID = 128
D_OUT = 64

NP = 10240
NC = 2
NS = 16
NW = NC * NS
K = 128
CH = 80
CHP = CH + 2
EP = NW * CHP * K
ROWS_PER_TILE = NP // NS

_mesh = functools.partial(
    plsc.VectorSubcoreMesh,
    core_axis_name="c", subcore_axis_name="s",
    num_cores=NC, num_subcores=NS)


def _make_deg_kernel():
    ZR = 64

    @functools.partial(
        pl.kernel,
        out_type=jax.ShapeDtypeStruct((NC, NP, 128), jnp.float32),
        mesh=_mesh(),
        scratch_types=[
            pltpu.VMEM((CHP, K), jnp.int32),
            pltpu.VMEM((K, 128), jnp.float32),
            pltpu.VMEM((ZR, 128), jnp.float32),
            pltpu.VMEM_SHARED((NP, 128), jnp.float32),
            pltpu.SemaphoreType.DMA,
        ],
    )
    def deg_kernel(dst_hbm, out_hbm, dst_v, ones_v, zb_v, acc_s, ssem):
        cid = lax.axis_index("c")
        sid = lax.axis_index("s")
        wid = sid * NC + cid
        pltpu.sync_copy(dst_hbm.at[wid], dst_v)
        one = jnp.ones((16,), jnp.float32)
        zero = jnp.zeros((16,), jnp.float32)
        for r in range(K):
            for c in range(128 // 16):
                ones_v[r, pl.ds(c * 16, 16)] = one
        for r in range(ZR):
            for c in range(128 // 16):
                zb_v[r, pl.ds(c * 16, 16)] = zero
        row0 = sid * ROWS_PER_TILE

        def zloop(t, carry):
            pltpu.sync_copy(zb_v, acc_s.at[pl.ds(row0 + t * ZR, ZR)])
            return carry

        lax.fori_loop(jnp.int32(0), jnp.int32(ROWS_PER_TILE // ZR), zloop,
                      jnp.int32(0))
        plsc.subcore_barrier()

        def chunk(j, carry):
            pltpu.sync_copy(ones_v, acc_s.at[dst_v.at[j]], add=True)
            return carry

        lax.fori_loop(jnp.int32(0), jnp.int32(CH), chunk, jnp.int32(0))
        plsc.subcore_barrier()
        pltpu.sync_copy(acc_s.at[pl.ds(row0, ROWS_PER_TILE)],
                        out_hbm.at[cid, pl.ds(row0, ROWS_PER_TILE)])

    return deg_kernel


def _make_agg_kernel(D):
    ZR = 64
    NG = 1
    KH = K // NG

    @functools.partial(
        pl.kernel,
        out_type=jax.ShapeDtypeStruct((NC, NP, D), jnp.float32),
        mesh=_mesh(),
        scratch_types=[
            pltpu.VMEM((CHP, K), jnp.int32),
            pltpu.VMEM((CHP, K), jnp.int32),
            pltpu.VMEM((K, D), jnp.float32),
            pltpu.VMEM((ZR, D), jnp.float32),
            pltpu.VMEM_SHARED((NP, D), jnp.float32),
            pltpu.SemaphoreType.DMA,
        ],
    )
    def agg_kernel(g_hbm, src_hbm, dst_hbm, out_hbm,
                   src_v, dst_v, rows_v, zb_v, acc_s, gsem):
        cid = lax.axis_index("c")
        sid = lax.axis_index("s")
        wid = sid * NC + cid
        pltpu.sync_copy(src_hbm.at[wid], src_v)
        pltpu.sync_copy(dst_hbm.at[wid], dst_v)
        zero = jnp.zeros((16,), jnp.float32)
        for r in range(ZR):
            for c in range(D // 16):
                zb_v[r, pl.ds(c * 16, 16)] = zero
        row0 = sid * ROWS_PER_TILE

        def zloop(t, carry):
            pltpu.sync_copy(zb_v, acc_s.at[pl.ds(row0 + t * ZR, ZR)])
            return carry

        lax.fori_loop(jnp.int32(0), jnp.int32(ROWS_PER_TILE // ZR),
                      zloop, jnp.int32(0))
        plsc.subcore_barrier()

        def chunk(j, carry):
            for h in range(NG):
                pltpu.async_copy(
                    g_hbm.at[src_v.at[j, pl.ds(h * KH, KH)]],
                    rows_v.at[pl.ds(h * KH, KH)], gsem)
            for h in range(NG):
                pltpu.make_async_copy(
                    g_hbm.at[src_v.at[j, pl.ds(h * KH, KH)]],
                    rows_v.at[pl.ds(h * KH, KH)], gsem).wait()
            pltpu.sync_copy(rows_v, acc_s.at[dst_v.at[j]], add=True)
            return carry

        lax.fori_loop(jnp.int32(0), jnp.int32(CH), chunk, jnp.int32(0))
        plsc.subcore_barrier()
        pltpu.sync_copy(acc_s.at[pl.ds(row0, ROWS_PER_TILE)],
                        out_hbm.at[cid, pl.ds(row0, ROWS_PER_TILE)])

    return agg_kernel


_deg_kernel = _make_deg_kernel()
_agg128 = _make_agg_kernel(D_HID)


def _tc_scale_in(x_pad, W1, degp):
    def body(x_ref, w_ref, degp_ref, g_ref, dis_ref):
        deg = degp_ref[0, :, 0:1] + degp_ref[1, :, 0:1] + 1.0
        dis = lax.rsqrt(deg)
        h = jnp.dot(x_ref[...], w_ref[...],
                    preferred_element_type=jnp.float32)
        g_ref[...] = h * dis
        dis_ref[...] = dis

    return pl.pallas_call(
        body,
        out_shape=(jax.ShapeDtypeStruct((NP, D_HID), jnp.float32),
                   jax.ShapeDtypeStruct((NP, 1), jnp.float32)),
    )(x_pad, W1, degp)


def _tc_mid(p1, g1, dis, b1, W2):
    def body(p_ref, g1_ref, dis_ref, b1_ref, w2_ref, g2_ref):
        dis = dis_ref[...]
        s = dis * (p_ref[0] + p_ref[1] + g1_ref[...]) + b1_ref[...]
        h = jnp.maximum(s, 0.0)
        g2_ref[...] = dis * jnp.dot(h, w2_ref[...],
                                    preferred_element_type=jnp.float32)

    return pl.pallas_call(
        body,
        out_shape=jax.ShapeDtypeStruct((NP, D_HID), jnp.float32),
    )(p1, g1, dis, b1, W2)


def _tc_out(p2, g2, dis, b2):
    def body(p_ref, g2_ref, dis_ref, b2_ref, z_ref):
        z_ref[...] = dis_ref[...] * (
            p_ref[0, :, :D_OUT] + p_ref[1, :, :D_OUT] + g2_ref[:, :D_OUT]
        ) + b2_ref[...]

    return pl.pallas_call(
        body,
        out_shape=jax.ShapeDtypeStruct((NP, D_OUT), jnp.float32),
    )(p2, g2, dis, b2)


def kernel(x, edge_index, W1, b1, W2, b2):
    ei = edge_index.astype(jnp.int32)
    pad = jnp.full((EP - N_EDGES,), NP - 1, jnp.int32)
    src3 = jnp.concatenate([ei[0], pad]).reshape(CHP, NW, K).transpose(1, 0, 2)
    dst3 = jnp.concatenate([ei[1], pad]).reshape(CHP, NW, K).transpose(1, 0, 2)
    x_pad = jnp.zeros((NP, D_IN), jnp.float32).at[:N_NODES].set(
        x.astype(jnp.float32))
    b1r = b1.astype(jnp.float32).reshape(1, D_HID)
    b2r = b2.astype(jnp.float32).reshape(1, D_OUT)

    W2p = jnp.zeros((D_HID, D_HID), jnp.float32).at[:, :D_OUT].set(
        W2.astype(jnp.float32))

    degp = _deg_kernel(dst3)
    g1, dis = _tc_scale_in(x_pad, W1.astype(jnp.float32), degp)
    p1 = _agg128(g1, src3, dst3)
    g2 = _tc_mid(p1, g1, dis, b1r, W2p)
    p2 = _agg128(g2, src3, dst3)
    z = _tc_out(p2, g2, dis, b2r)
    return z[:N_NODES].astype(jnp.float64)


# speedup vs baseline: 2.4410x; 2.0279x over previous
import functools

import jax
import jax.numpy as jnp
from jax import lax
from jax.experimental import pallas as pl
from jax.experimental.pallas import tpu as pltpu
from jax.experimental.pallas import tpu_sc as plsc

N_NODES = 10000
N_EDGES = 320000
D_IN = 128
D_HID = 128
D_OUT = 64

NP = 10240
NC = 2
NS = 16
NW = NC * NS
K = 128
CH = 80
CHP = CH + 2
EP = NW * CHP * K
ROWS_PER_TILE = NP // NS

_mesh = functools.partial(
    plsc.VectorSubcoreMesh,
    core_axis_name="c", subcore_axis_name="s",
    num_cores=NC, num_subcores=NS)


def _make_deg_kernel():
    ZR = 64

    @functools.partial(
        pl.kernel,
        out_type=jax.ShapeDtypeStruct((NC, NP, 128), jnp.float32),
        mesh=_mesh(),
        scratch_types=[
            pltpu.VMEM((CHP, K), jnp.int32),
            pltpu.VMEM((K, 128), jnp.float32),
            pltpu.VMEM((ZR, 128), jnp.float32),
            pltpu.VMEM_SHARED((NP, 128), jnp.float32),
            pltpu.SemaphoreType.DMA,
        ],
    )
    def deg_kernel(dst_hbm, out_hbm, dst_v, ones_v, zb_v, acc_s, ssem):
        cid = lax.axis_index("c")
        sid = lax.axis_index("s")
        wid = sid * NC + cid
        pltpu.sync_copy(dst_hbm.at[wid], dst_v)
        one = jnp.ones((16,), jnp.float32)
        zero = jnp.zeros((16,), jnp.float32)
        for r in range(K):
            for c in range(128 // 16):
                ones_v[r, pl.ds(c * 16, 16)] = one
        for r in range(ZR):
            for c in range(128 // 16):
                zb_v[r, pl.ds(c * 16, 16)] = zero
        row0 = sid * ROWS_PER_TILE

        def zloop(t, carry):
            pltpu.sync_copy(zb_v, acc_s.at[pl.ds(row0 + t * ZR, ZR)])
            return carry

        lax.fori_loop(jnp.int32(0), jnp.int32(ROWS_PER_TILE // ZR), zloop,
                      jnp.int32(0))
        plsc.subcore_barrier()

        def chunk(j, carry):
            pltpu.sync_copy(ones_v, acc_s.at[dst_v.at[j]], add=True)
            return carry

        lax.fori_loop(jnp.int32(0), jnp.int32(CH), chunk, jnp.int32(0))
        plsc.subcore_barrier()
        pltpu.sync_copy(acc_s.at[pl.ds(row0, ROWS_PER_TILE)],
                        out_hbm.at[cid, pl.ds(row0, ROWS_PER_TILE)])

    return deg_kernel


def _make_agg_kernel(D):
    ZR = 64
    NG = 1
    KH = K // NG

    @functools.partial(
        pl.kernel,
        out_type=jax.ShapeDtypeStruct((NC, NP, D), jnp.float32),
        mesh=_mesh(),
        scratch_types=[
            pltpu.VMEM((CHP, K), jnp.int32),
            pltpu.VMEM((CHP, K), jnp.int32),
            pltpu.VMEM((K, D), jnp.float32),
            pltpu.VMEM((ZR, D), jnp.float32),
            pltpu.VMEM_SHARED((NP, D), jnp.float32),
            pltpu.SemaphoreType.DMA,
        ],
    )
    def agg_kernel(g_hbm, src_hbm, dst_hbm, out_hbm,
                   src_v, dst_v, rows_v, zb_v, acc_s, gsem):
        cid = lax.axis_index("c")
        sid = lax.axis_index("s")
        wid = sid * NC + cid
        pltpu.sync_copy(src_hbm.at[wid], src_v)
        pltpu.sync_copy(dst_hbm.at[wid], dst_v)
        zero = jnp.zeros((16,), jnp.float32)
        for r in range(ZR):
            for c in range(D // 16):
                zb_v[r, pl.ds(c * 16, 16)] = zero
        row0 = sid * ROWS_PER_TILE

        def zloop(t, carry):
            pltpu.sync_copy(zb_v, acc_s.at[pl.ds(row0 + t * ZR, ZR)])
            return carry

        lax.fori_loop(jnp.int32(0), jnp.int32(ROWS_PER_TILE // ZR),
                      zloop, jnp.int32(0))
        plsc.subcore_barrier()

        def chunk(j, carry):
            for h in range(NG):
                pltpu.async_copy(
                    g_hbm.at[src_v.at[j, pl.ds(h * KH, KH)]],
                    rows_v.at[pl.ds(h * KH, KH)], gsem)
            for h in range(NG):
                pltpu.make_async_copy(
                    g_hbm.at[src_v.at[j, pl.ds(h * KH, KH)]],
                    rows_v.at[pl.ds(h * KH, KH)], gsem).wait()
            pltpu.sync_copy(rows_v, acc_s.at[dst_v.at[j]], add=True)
            return carry

        lax.fori_loop(jnp.int32(0), jnp.int32(CH), chunk, jnp.int32(0))
        plsc.subcore_barrier()
        pltpu.sync_copy(acc_s.at[pl.ds(row0, ROWS_PER_TILE)],
                        out_hbm.at[cid, pl.ds(row0, ROWS_PER_TILE)])

    return agg_kernel


_deg_kernel = _make_deg_kernel()
_agg128 = _make_agg_kernel(D_HID)


def _tc_scale_in(x_pad, W1, degp):
    def body(x_ref, w_ref, degp_ref, g_ref, dis_ref):
        deg = degp_ref[0, :, 0:1] + degp_ref[1, :, 0:1] + 1.0
        dis = lax.rsqrt(deg)
        h = jnp.dot(x_ref[...], w_ref[...],
                    preferred_element_type=jnp.float32)
        g_ref[...] = h * dis
        dis_ref[...] = dis

    return pl.pallas_call(
        body,
        out_shape=(jax.ShapeDtypeStruct((NP, D_HID), jnp.float32),
                   jax.ShapeDtypeStruct((NP, 1), jnp.float32)),
    )(x_pad, W1, degp)


def _tc_mid(p1, g1, dis, b1, W2):
    def body(p_ref, g1_ref, dis_ref, b1_ref, w2_ref, g2_ref):
        dis = dis_ref[...]
        s = dis * (p_ref[0] + p_ref[1] + g1_ref[...]) + b1_ref[...]
        h = jnp.maximum(s, 0.0)
        g2_ref[...] = dis * jnp.dot(h, w2_ref[...],
                                    preferred_element_type=jnp.float32)

    return pl.pallas_call(
        body,
        out_shape=jax.ShapeDtypeStruct((NP, D_HID), jnp.float32),
    )(p1, g1, dis, b1, W2)


def _tc_out(p2, g2, dis, b2):
    def body(p_ref, g2_ref, dis_ref, b2_ref, z_ref):
        z_ref[...] = dis_ref[...] * (
            p_ref[0, :, :D_OUT] + p_ref[1, :, :D_OUT] + g2_ref[:, :D_OUT]
        ) + b2_ref[...]

    return pl.pallas_call(
        body,
        out_shape=jax.ShapeDtypeStruct((NP, D_OUT), jnp.float32),
    )(p2, g2, dis, b2)


def kernel(x, edge_index, W1, b1, W2, b2):
    ei = edge_index.astype(jnp.int32)
    pad = N_NODES + jnp.arange(EP - N_EDGES, dtype=jnp.int32) % (NP - N_NODES)
    src3 = jnp.concatenate([ei[0], pad]).reshape(CHP, NW, K).transpose(1, 0, 2)
    dst3 = jnp.concatenate([ei[1], pad]).reshape(CHP, NW, K).transpose(1, 0, 2)
    x_pad = jnp.zeros((NP, D_IN), jnp.float32).at[:N_NODES].set(
        x.astype(jnp.float32))
    b1r = b1.astype(jnp.float32).reshape(1, D_HID)
    b2r = b2.astype(jnp.float32).reshape(1, D_OUT)

    W2p = jnp.zeros((D_HID, D_HID), jnp.float32).at[:, :D_OUT].set(
        W2.astype(jnp.float32))

    degp = _deg_kernel(dst3)
    g1, dis = _tc_scale_in(x_pad, W1.astype(jnp.float32), degp)
    p1 = _agg128(g1, src3, dst3)
    g2 = _tc_mid(p1, g1, dis, b1r, W2p)
    p2 = _agg128(g2, src3, dst3)
    z = _tc_out(p2, g2, dis, b2r)
    return z[:N_NODES].astype(jnp.float64)


# speedup vs baseline: 2.6619x; 1.0905x over previous
import functools

import jax
import jax.numpy as jnp
from jax import lax
from jax.experimental import pallas as pl
from jax.experimental.pallas import tpu as pltpu
from jax.experimental.pallas import tpu_sc as plsc

N_NODES = 10000
N_EDGES = 320000
D_IN = 128
D_HID = 128
D_OUT = 64

NP = 10240
NC = 2
NS = 16
NW = NC * NS
K = 128
CH = 80
CHP = CH + 2
EP = NW * CHP * K
ROWS_PER_TILE = NP // NS

_mesh = functools.partial(
    plsc.VectorSubcoreMesh,
    core_axis_name="c", subcore_axis_name="s",
    num_cores=NC, num_subcores=NS)


def _make_deg_kernel():
    ZR = 64

    @functools.partial(
        pl.kernel,
        out_type=jax.ShapeDtypeStruct((NC, NP, 128), jnp.float32),
        mesh=_mesh(),
        scratch_types=[
            pltpu.VMEM((CHP, K), jnp.int32),
            pltpu.VMEM((K, 128), jnp.float32),
            pltpu.VMEM((ZR, 128), jnp.float32),
            pltpu.VMEM_SHARED((NP, 128), jnp.float32),
            pltpu.SemaphoreType.DMA,
        ],
    )
    def deg_kernel(dst_hbm, out_hbm, dst_v, ones_v, zb_v, acc_s, ssem):
        cid = lax.axis_index("c")
        sid = lax.axis_index("s")
        wid = sid * NC + cid
        pltpu.sync_copy(dst_hbm.at[wid], dst_v)
        one = jnp.ones((16,), jnp.float32)
        zero = jnp.zeros((16,), jnp.float32)
        for r in range(K):
            for c in range(128 // 16):
                ones_v[r, pl.ds(c * 16, 16)] = one
        for r in range(ZR):
            for c in range(128 // 16):
                zb_v[r, pl.ds(c * 16, 16)] = zero
        row0 = sid * ROWS_PER_TILE

        def zloop(t, carry):
            pltpu.sync_copy(zb_v, acc_s.at[pl.ds(row0 + t * ZR, ZR)])
            return carry

        lax.fori_loop(jnp.int32(0), jnp.int32(ROWS_PER_TILE // ZR), zloop,
                      jnp.int32(0))
        plsc.subcore_barrier()

        def chunk(j, carry):
            pltpu.sync_copy(ones_v, acc_s.at[dst_v.at[j]], add=True)
            return carry

        lax.fori_loop(jnp.int32(0), jnp.int32(CH), chunk, jnp.int32(0))
        plsc.subcore_barrier()
        pltpu.sync_copy(acc_s.at[pl.ds(row0, ROWS_PER_TILE)],
                        out_hbm.at[cid, pl.ds(row0, ROWS_PER_TILE)])

    return deg_kernel


def _make_agg_kernel(D, tc_tiling=True):
    ZR = 64
    NG = 1
    KH = K // NG

    @functools.partial(
        pl.kernel,
        out_type=jax.ShapeDtypeStruct((NC, NP, D), jnp.float32),
        mesh=_mesh(),
        compiler_params=pltpu.CompilerParams(use_tc_tiling_on_sc=tc_tiling),
        scratch_types=[
            pltpu.VMEM((CHP, K), jnp.int32),
            pltpu.VMEM((CHP, K), jnp.int32),
            pltpu.VMEM((K, D), jnp.float32),
            pltpu.VMEM((ZR, D), jnp.float32),
            pltpu.VMEM_SHARED((NP, D), jnp.float32),
            pltpu.SemaphoreType.DMA,
        ],
    )
    def agg_kernel(g_hbm, src_hbm, dst_hbm, out_hbm,
                   src_v, dst_v, rows_v, zb_v, acc_s, gsem):
        cid = lax.axis_index("c")
        sid = lax.axis_index("s")
        wid = sid * NC + cid
        pltpu.sync_copy(src_hbm.at[wid], src_v)
        pltpu.sync_copy(dst_hbm.at[wid], dst_v)
        zero = jnp.zeros((16,), jnp.float32)
        for r in range(ZR):
            for c in range(D // 16):
                zb_v[r, pl.ds(c * 16, 16)] = zero
        row0 = sid * ROWS_PER_TILE

        def zloop(t, carry):
            pltpu.sync_copy(zb_v, acc_s.at[pl.ds(row0 + t * ZR, ZR)])
            return carry

        lax.fori_loop(jnp.int32(0), jnp.int32(ROWS_PER_TILE // ZR),
                      zloop, jnp.int32(0))
        plsc.subcore_barrier()

        def chunk(j, carry):
            for h in range(NG):
                pltpu.async_copy(
                    g_hbm.at[src_v.at[j, pl.ds(h * KH, KH)]],
                    rows_v.at[pl.ds(h * KH, KH)], gsem)
            for h in range(NG):
                pltpu.make_async_copy(
                    g_hbm.at[src_v.at[j, pl.ds(h * KH, KH)]],
                    rows_v.at[pl.ds(h * KH, KH)], gsem).wait()
            pltpu.sync_copy(rows_v, acc_s.at[dst_v.at[j]], add=True)
            return carry

        lax.fori_loop(jnp.int32(0), jnp.int32(CH), chunk, jnp.int32(0))
        plsc.subcore_barrier()
        pltpu.sync_copy(acc_s.at[pl.ds(row0, ROWS_PER_TILE)],
                        out_hbm.at[cid, pl.ds(row0, ROWS_PER_TILE)])

    return agg_kernel


_deg_kernel = _make_deg_kernel()
_agg128 = _make_agg_kernel(D_HID)
_agg64 = _make_agg_kernel(D_OUT, tc_tiling=False)


def _tc_scale_in(x_pad, W1, degp):
    def body(x_ref, w_ref, degp_ref, g_ref, dis_ref):
        deg = degp_ref[0, :, 0:1] + degp_ref[1, :, 0:1] + 1.0
        dis = lax.rsqrt(deg)
        h = jnp.dot(x_ref[...], w_ref[...],
                    preferred_element_type=jnp.float32)
        g_ref[...] = h * dis
        dis_ref[...] = dis

    return pl.pallas_call(
        body,
        out_shape=(jax.ShapeDtypeStruct((NP, D_HID), jnp.float32),
                   jax.ShapeDtypeStruct((NP, 1), jnp.float32)),
    )(x_pad, W1, degp)


def _tc_mid(p1, g1, dis, b1, W2):
    def body(p_ref, g1_ref, dis_ref, b1_ref, w2_ref, g2_ref):
        dis = dis_ref[...]
        s = dis * (p_ref[0] + p_ref[1] + g1_ref[...]) + b1_ref[...]
        h = jnp.maximum(s, 0.0)
        g2_ref[...] = dis * jnp.dot(h, w2_ref[...],
                                    preferred_element_type=jnp.float32)

    return pl.pallas_call(
        body,
        out_shape=jax.ShapeDtypeStruct((NP, D_OUT), jnp.float32),
    )(p1, g1, dis, b1, W2)


def _tc_out(p2, g2, dis, b2):
    def body(p_ref, g2_ref, dis_ref, b2_ref, z_ref):
        z_ref[...] = dis_ref[...] * (
            p_ref[0] + p_ref[1] + g2_ref[...]
        ) + b2_ref[...]

    return pl.pallas_call(
        body,
        out_shape=jax.ShapeDtypeStruct((NP, D_OUT), jnp.float32),
    )(p2, g2, dis, b2)


def kernel(x, edge_index, W1, b1, W2, b2):
    ei = edge_index.astype(jnp.int32)
    pad = N_NODES + jnp.arange(EP - N_EDGES, dtype=jnp.int32) % (NP - N_NODES)
    src3 = jnp.concatenate([ei[0], pad]).reshape(CHP, NW, K).transpose(1, 0, 2)
    dst3 = jnp.concatenate([ei[1], pad]).reshape(CHP, NW, K).transpose(1, 0, 2)
    x_pad = jnp.zeros((NP, D_IN), jnp.float32).at[:N_NODES].set(
        x.astype(jnp.float32))
    b1r = b1.astype(jnp.float32).reshape(1, D_HID)
    b2r = b2.astype(jnp.float32).reshape(1, D_OUT)

    degp = _deg_kernel(dst3)
    g1, dis = _tc_scale_in(x_pad, W1.astype(jnp.float32), degp)
    p1 = _agg128(g1, src3, dst3)
    g2 = _tc_mid(p1, g1, dis, b1r, W2.astype(jnp.float32))
    p2 = _agg64(g2, src3, dst3)
    z = _tc_out(p2, g2, dis, b2r)
    return z[:N_NODES].astype(jnp.float64)


# speedup vs baseline: 2.9709x; 1.1161x over previous
import functools

import jax
import jax.numpy as jnp
from jax import lax
from jax.experimental import pallas as pl
from jax.experimental.pallas import tpu as pltpu
from jax.experimental.pallas import tpu_sc as plsc

N_NODES = 10000
N_EDGES = 320000
D_IN = 128
D_HID = 128
D_OUT = 64

NP = 10240
NC = 2
NS = 16
NW = NC * NS
K = 128
CH = 80
CHP = CH + 2
EP = NW * CHP * K
ROWS_PER_TILE = NP // NS

_mesh = functools.partial(
    plsc.VectorSubcoreMesh,
    core_axis_name="c", subcore_axis_name="s",
    num_cores=NC, num_subcores=NS)


def _make_deg_kernel():
    ZR = 64
    DW = 16

    @functools.partial(
        pl.kernel,
        out_type=jax.ShapeDtypeStruct((NC, NP, DW), jnp.float32),
        mesh=_mesh(),
        compiler_params=pltpu.CompilerParams(use_tc_tiling_on_sc=False),
        scratch_types=[
            pltpu.VMEM((CHP, K), jnp.int32),
            pltpu.VMEM((K, DW), jnp.float32),
            pltpu.VMEM((ZR, DW), jnp.float32),
            pltpu.VMEM_SHARED((NP, DW), jnp.float32),
            pltpu.SemaphoreType.DMA,
        ],
    )
    def deg_kernel(dst_hbm, out_hbm, dst_v, ones_v, zb_v, acc_s, ssem):
        cid = lax.axis_index("c")
        sid = lax.axis_index("s")
        wid = sid * NC + cid
        pltpu.sync_copy(dst_hbm.at[wid], dst_v)
        one = jnp.ones((16,), jnp.float32)
        zero = jnp.zeros((16,), jnp.float32)
        for r in range(K):
            for c in range(DW // 16):
                ones_v[r, pl.ds(c * 16, 16)] = one
        for r in range(ZR):
            for c in range(DW // 16):
                zb_v[r, pl.ds(c * 16, 16)] = zero
        row0 = sid * ROWS_PER_TILE

        def zloop(t, carry):
            pltpu.sync_copy(zb_v, acc_s.at[pl.ds(row0 + t * ZR, ZR)])
            return carry

        lax.fori_loop(jnp.int32(0), jnp.int32(ROWS_PER_TILE // ZR), zloop,
                      jnp.int32(0))
        plsc.subcore_barrier()

        pltpu.async_copy(ones_v, acc_s.at[dst_v.at[jnp.int32(0)]], ssem,
                         add=True)

        def chunk(j, carry):
            pltpu.async_copy(ones_v, acc_s.at[dst_v.at[j + 1]], ssem,
                             add=True)
            pltpu.make_async_copy(ones_v, acc_s.at[dst_v.at[j]], ssem).wait()
            return carry

        lax.fori_loop(jnp.int32(0), jnp.int32(CH - 1), chunk, jnp.int32(0))
        pltpu.make_async_copy(ones_v, acc_s.at[dst_v.at[jnp.int32(CH - 1)]],
                              ssem).wait()
        plsc.subcore_barrier()
        pltpu.sync_copy(acc_s.at[pl.ds(row0, ROWS_PER_TILE)],
                        out_hbm.at[cid, pl.ds(row0, ROWS_PER_TILE)])

    return deg_kernel


def _make_agg_kernel(D, tc_tiling=True):
    ZR = 64
    NG = 1
    KH = K // NG

    @functools.partial(
        pl.kernel,
        out_type=jax.ShapeDtypeStruct((NC, NP, D), jnp.float32),
        mesh=_mesh(),
        compiler_params=pltpu.CompilerParams(use_tc_tiling_on_sc=tc_tiling),
        scratch_types=[
            pltpu.VMEM((CHP, K), jnp.int32),
            pltpu.VMEM((CHP, K), jnp.int32),
            pltpu.VMEM((K, D), jnp.float32),
            pltpu.VMEM((ZR, D), jnp.float32),
            pltpu.VMEM_SHARED((NP, D), jnp.float32),
            pltpu.SemaphoreType.DMA,
        ],
    )
    def agg_kernel(g_hbm, src_hbm, dst_hbm, out_hbm,
                   src_v, dst_v, rows_v, zb_v, acc_s, gsem):
        cid = lax.axis_index("c")
        sid = lax.axis_index("s")
        wid = sid * NC + cid
        pltpu.sync_copy(src_hbm.at[wid], src_v)
        pltpu.sync_copy(dst_hbm.at[wid], dst_v)
        zero = jnp.zeros((16,), jnp.float32)
        for r in range(ZR):
            for c in range(D // 16):
                zb_v[r, pl.ds(c * 16, 16)] = zero
        row0 = sid * ROWS_PER_TILE

        def zloop(t, carry):
            pltpu.sync_copy(zb_v, acc_s.at[pl.ds(row0 + t * ZR, ZR)])
            return carry

        lax.fori_loop(jnp.int32(0), jnp.int32(ROWS_PER_TILE // ZR),
                      zloop, jnp.int32(0))
        plsc.subcore_barrier()

        def chunk(j, carry):
            for h in range(NG):
                pltpu.async_copy(
                    g_hbm.at[src_v.at[j, pl.ds(h * KH, KH)]],
                    rows_v.at[pl.ds(h * KH, KH)], gsem)
            for h in range(NG):
                pltpu.make_async_copy(
                    g_hbm.at[src_v.at[j, pl.ds(h * KH, KH)]],
                    rows_v.at[pl.ds(h * KH, KH)], gsem).wait()
            pltpu.sync_copy(rows_v, acc_s.at[dst_v.at[j]], add=True)
            return carry

        lax.fori_loop(jnp.int32(0), jnp.int32(CH), chunk, jnp.int32(0))
        plsc.subcore_barrier()
        pltpu.sync_copy(acc_s.at[pl.ds(row0, ROWS_PER_TILE)],
                        out_hbm.at[cid, pl.ds(row0, ROWS_PER_TILE)])

    return agg_kernel


_deg_kernel = _make_deg_kernel()
_agg128 = _make_agg_kernel(D_HID)
_agg64 = _make_agg_kernel(D_OUT, tc_tiling=False)


def _tc_scale_in(x_pad, W1, degp):
    def body(x_ref, w_ref, degp_ref, g_ref, dis_ref):
        deg = degp_ref[0, :, 0:1] + degp_ref[1, :, 0:1] + 1.0
        dis = lax.rsqrt(deg)
        h = jnp.dot(x_ref[...], w_ref[...],
                    preferred_element_type=jnp.float32)
        g_ref[...] = h * dis
        dis_ref[...] = dis

    return pl.pallas_call(
        body,
        out_shape=(jax.ShapeDtypeStruct((NP, D_HID), jnp.float32),
                   jax.ShapeDtypeStruct((NP, 1), jnp.float32)),
    )(x_pad, W1, degp)


def _tc_mid(p1, g1, dis, b1, W2):
    def body(p_ref, g1_ref, dis_ref, b1_ref, w2_ref, g2_ref):
        dis = dis_ref[...]
        s = dis * (p_ref[0] + p_ref[1] + g1_ref[...]) + b1_ref[...]
        h = jnp.maximum(s, 0.0)
        g2_ref[...] = dis * jnp.dot(h, w2_ref[...],
                                    preferred_element_type=jnp.float32)

    return pl.pallas_call(
        body,
        out_shape=jax.ShapeDtypeStruct((NP, D_OUT), jnp.float32),
    )(p1, g1, dis, b1, W2)


def _tc_out(p2, g2, dis, b2):
    def body(p_ref, g2_ref, dis_ref, b2_ref, z_ref):
        z_ref[...] = dis_ref[...] * (
            p_ref[0] + p_ref[1] + g2_ref[...]
        ) + b2_ref[...]

    return pl.pallas_call(
        body,
        out_shape=jax.ShapeDtypeStruct((NP, D_OUT), jnp.float32),
    )(p2, g2, dis, b2)


def kernel(x, edge_index, W1, b1, W2, b2):
    ei = edge_index.astype(jnp.int32)
    pad = N_NODES + jnp.arange(EP - N_EDGES, dtype=jnp.int32) % (NP - N_NODES)
    src3 = jnp.concatenate([ei[0], pad]).reshape(CHP, NW, K).transpose(1, 0, 2)
    dst3 = jnp.concatenate([ei[1], pad]).reshape(CHP, NW, K).transpose(1, 0, 2)
    x_pad = jnp.zeros((NP, D_IN), jnp.float32).at[:N_NODES].set(
        x.astype(jnp.float32))
    b1r = b1.astype(jnp.float32).reshape(1, D_HID)
    b2r = b2.astype(jnp.float32).reshape(1, D_OUT)

    degp = _deg_kernel(dst3)
    g1, dis = _tc_scale_in(x_pad, W1.astype(jnp.float32), degp)
    p1 = _agg128(g1, src3, dst3)
    g2 = _tc_mid(p1, g1, dis, b1r, W2.astype(jnp.float32))
    p2 = _agg64(g2, src3, dst3)
    z = _tc_out(p2, g2, dis, b2r)
    return z[:N_NODES].astype(jnp.float64)


# speedup vs baseline: 3.5649x; 1.1999x over previous
import functools

import jax
import jax.numpy as jnp
from jax import lax
from jax.experimental import pallas as pl
from jax.experimental.pallas import tpu as pltpu
from jax.experimental.pallas import tpu_sc as plsc

N_NODES = 10000
N_EDGES = 320000
D_IN = 128
D_HID = 128
D_OUT = 64

NP = 10240
NC = 2
NS = 16
NW = NC * NS
K = 128
CH = 80
CHP = CH + 2
EP = NW * CHP * K
ROWS_PER_TILE = NP // NS

_mesh = functools.partial(
    plsc.VectorSubcoreMesh,
    core_axis_name="c", subcore_axis_name="s",
    num_cores=NC, num_subcores=NS)


def _make_deg_kernel():
    ZR = 64
    DW = 16

    @functools.partial(
        pl.kernel,
        out_type=jax.ShapeDtypeStruct((NC, NP, DW), jnp.float32),
        mesh=_mesh(),
        compiler_params=pltpu.CompilerParams(use_tc_tiling_on_sc=False),
        scratch_types=[
            pltpu.VMEM((CHP, K), jnp.int32),
            pltpu.VMEM((K, DW), jnp.float32),
            pltpu.VMEM((ZR, DW), jnp.float32),
            pltpu.VMEM_SHARED((NP, DW), jnp.float32),
            pltpu.SemaphoreType.DMA,
        ],
    )
    def deg_kernel(dst_hbm, out_hbm, dst_v, ones_v, zb_v, acc_s, ssem):
        cid = lax.axis_index("c")
        sid = lax.axis_index("s")
        wid = sid * NC + cid
        pltpu.sync_copy(dst_hbm.at[wid], dst_v)
        one = jnp.ones((16,), jnp.float32)
        zero = jnp.zeros((16,), jnp.float32)
        for r in range(K):
            for c in range(DW // 16):
                ones_v[r, pl.ds(c * 16, 16)] = one
        for r in range(ZR):
            for c in range(DW // 16):
                zb_v[r, pl.ds(c * 16, 16)] = zero
        row0 = sid * ROWS_PER_TILE

        def zloop(t, carry):
            pltpu.sync_copy(zb_v, acc_s.at[pl.ds(row0 + t * ZR, ZR)])
            return carry

        lax.fori_loop(jnp.int32(0), jnp.int32(ROWS_PER_TILE // ZR), zloop,
                      jnp.int32(0))
        plsc.subcore_barrier()

        pltpu.async_copy(ones_v, acc_s.at[dst_v.at[jnp.int32(0)]], ssem,
                         add=True)

        def chunk(j, carry):
            pltpu.async_copy(ones_v, acc_s.at[dst_v.at[j + 1]], ssem,
                             add=True)
            pltpu.make_async_copy(ones_v, acc_s.at[dst_v.at[j]], ssem).wait()
            return carry

        lax.fori_loop(jnp.int32(0), jnp.int32(CH - 1), chunk, jnp.int32(0))
        pltpu.make_async_copy(ones_v, acc_s.at[dst_v.at[jnp.int32(CH - 1)]],
                              ssem).wait()
        plsc.subcore_barrier()
        pltpu.sync_copy(acc_s.at[pl.ds(row0, ROWS_PER_TILE)],
                        out_hbm.at[cid, pl.ds(row0, ROWS_PER_TILE)])

    return deg_kernel


def _make_agg_kernel(D, tc_tiling=True):
    @functools.partial(
        pl.kernel,
        out_type=jax.ShapeDtypeStruct((NC, NP, D), jnp.float32),
        mesh=_mesh(),
        compiler_params=pltpu.CompilerParams(use_tc_tiling_on_sc=tc_tiling),
        scratch_types=[
            pltpu.VMEM((CHP, K), jnp.int32),
            pltpu.VMEM((K,), jnp.int32),
            pltpu.VMEM((K,), jnp.int32),
            pltpu.VMEM((K,), jnp.int32),
            pltpu.VMEM((K,), jnp.int32),
            pltpu.VMEM((K, D), jnp.float32),
            pltpu.VMEM((K, D), jnp.float32),
            pltpu.VMEM_SHARED((NP, D), jnp.float32),
            pltpu.SemaphoreType.DMA,
            pltpu.SemaphoreType.DMA,
            pltpu.SemaphoreType.DMA,
        ],
    )
    def agg_kernel(g_hbm, pk_hbm, out_hbm,
                   pk_v, sidx0, sidx1, didx0, didx1, rows0, rows1, acc_s,
                   gsem, ssem0, ssem1):
        cid = lax.axis_index("c")
        sid = lax.axis_index("s")
        wid = sid * NC + cid
        sidx = (sidx0, sidx1)
        didx = (didx0, didx1)
        rows = (rows0, rows1)
        ssem = (ssem0, ssem1)
        pltpu.sync_copy(pk_hbm.at[wid], pk_v)
        zero = jnp.zeros((16,), jnp.float32)

        def zrow(r, carry):
            for c in range(D // 16):
                rows0[r, pl.ds(c * 16, 16)] = zero
            return carry

        lax.fori_loop(jnp.int32(0), jnp.int32(K), zrow, jnp.int32(0))
        row0 = sid * ROWS_PER_TILE

        def zloop(t, carry):
            pltpu.sync_copy(rows0, acc_s.at[pl.ds(row0 + t * K, K)])
            return carry

        lax.fori_loop(jnp.int32(0), jnp.int32(ROWS_PER_TILE // K), zloop,
                      jnp.int32(0))
        plsc.subcore_barrier()

        mask = jnp.full((16,), 16383, jnp.int32)
        sh = jnp.full((16,), 14, jnp.int32)

        def unpack(j, e):
            for c in range(K // 16):
                v = pk_v[j, pl.ds(c * 16, 16)]
                sidx[e][pl.ds(c * 16, 16)] = jnp.bitwise_and(v, mask)
                didx[e][pl.ds(c * 16, 16)] = jnp.right_shift(v, sh)

        unpack(jnp.int32(0), 0)
        pltpu.async_copy(g_hbm.at[sidx[0]], rows[0], gsem)
        unpack(jnp.int32(1), 1)
        pltpu.make_async_copy(g_hbm.at[sidx[0]], rows[0], gsem).wait()

        def pipestep(u, carry):
            for e in range(2):
                t = u * 2 + e
                o = 1 - e
                pltpu.async_copy(rows[e], acc_s.at[didx[e]], ssem[e],
                                 add=True)
                pltpu.async_copy(g_hbm.at[sidx[o]], rows[o], gsem)
                pltpu.make_async_copy(rows[e], acc_s.at[didx[e]],
                                      ssem[e]).wait()
                unpack(t + 2, e)
                pltpu.make_async_copy(g_hbm.at[sidx[o]], rows[o],
                                      gsem).wait()
            return carry

        lax.fori_loop(jnp.int32(0), jnp.int32(CH // 2), pipestep,
                      jnp.int32(0))
        plsc.subcore_barrier()
        pltpu.sync_copy(acc_s.at[pl.ds(row0, ROWS_PER_TILE)],
                        out_hbm.at[cid, pl.ds(row0, ROWS_PER_TILE)])

    return agg_kernel


_deg_kernel = _make_deg_kernel()
_agg128 = _make_agg_kernel(D_HID)
_agg64 = _make_agg_kernel(D_OUT, tc_tiling=False)


def _tc_scale_in(x_pad, W1, degp):
    def body(x_ref, w_ref, degp_ref, g_ref, dis_ref):
        deg = degp_ref[0, :, 0:1] + degp_ref[1, :, 0:1] + 1.0
        dis = lax.rsqrt(deg)
        h = jnp.dot(x_ref[...], w_ref[...],
                    preferred_element_type=jnp.float32)
        g_ref[...] = h * dis
        dis_ref[...] = dis

    return pl.pallas_call(
        body,
        out_shape=(jax.ShapeDtypeStruct((NP, D_HID), jnp.float32),
                   jax.ShapeDtypeStruct((NP, 1), jnp.float32)),
    )(x_pad, W1, degp)


def _tc_mid(p1, g1, dis, b1, W2):
    def body(p_ref, g1_ref, dis_ref, b1_ref, w2_ref, g2_ref):
        dis = dis_ref[...]
        s = dis * (p_ref[0] + p_ref[1] + g1_ref[...]) + b1_ref[...]
        h = jnp.maximum(s, 0.0)
        g2_ref[...] = dis * jnp.dot(h, w2_ref[...],
                                    preferred_element_type=jnp.float32)

    return pl.pallas_call(
        body,
        out_shape=jax.ShapeDtypeStruct((NP, D_OUT), jnp.float32),
    )(p1, g1, dis, b1, W2)


def _tc_out(p2, g2, dis, b2):
    def body(p_ref, g2_ref, dis_ref, b2_ref, z_ref):
        z_ref[...] = dis_ref[...] * (
            p_ref[0] + p_ref[1] + g2_ref[...]
        ) + b2_ref[...]

    return pl.pallas_call(
        body,
        out_shape=jax.ShapeDtypeStruct((NP, D_OUT), jnp.float32),
    )(p2, g2, dis, b2)


def kernel(x, edge_index, W1, b1, W2, b2):
    ei = edge_index.astype(jnp.int32)
    pad = N_NODES + jnp.arange(EP - N_EDGES, dtype=jnp.int32) % (NP - N_NODES)
    src3 = jnp.concatenate([ei[0], pad]).reshape(CHP, NW, K).transpose(1, 0, 2)
    dst3 = jnp.concatenate([ei[1], pad]).reshape(CHP, NW, K).transpose(1, 0, 2)
    x_pad = jnp.zeros((NP, D_IN), jnp.float32).at[:N_NODES].set(
        x.astype(jnp.float32))
    b1r = b1.astype(jnp.float32).reshape(1, D_HID)
    b2r = b2.astype(jnp.float32).reshape(1, D_OUT)

    pk3 = src3 | (dst3 << 14)

    degp = _deg_kernel(dst3)
    g1, dis = _tc_scale_in(x_pad, W1.astype(jnp.float32), degp)
    p1 = _agg128(g1, pk3)
    g2 = _tc_mid(p1, g1, dis, b1r, W2.astype(jnp.float32))
    p2 = _agg64(g2, pk3)
    z = _tc_out(p2, g2, dis, b2r)
    return z[:N_NODES].astype(jnp.float64)
